# baseline (speedup 1.0000x reference)
import functools

import jax
import jax.numpy as jnp
from jax.experimental import pallas as pl

N = 10000
E = 320000
F_IN = 128
H = 256
D_E = 4
KAN_OUT = 100
GS = 5
SO = 3


def _make_grid(in_features):
    h = 2.0 / GS
    g = jnp.arange(-SO, GS + SO + 1, dtype=jnp.float32) * h - 1.0
    return jnp.tile(g[None, :], (in_features, 1))


def _b_splines(x, grid):
    x = x[..., None]
    bases = ((x >= grid[:, :-1]) & (x < grid[:, 1:])).astype(x.dtype)
    for k in range(1, SO + 1):
        left = (x - grid[:, : -(k + 1)]) / (grid[:, k:-1] - grid[:, : -(k + 1)])
        right = (grid[:, k + 1:] - x) / (grid[:, k + 1:] - grid[:, 1:-k])
        bases = left * bases[..., :-1] + right * bases[..., 1:]
    return bases


def _kan_linear(x, base_w, spline_w, grid):
    base = jax.nn.silu(x) @ base_w.T
    sp = _b_splines(x, grid).reshape(x.shape[0], -1) @ spline_w.reshape(spline_w.shape[0], -1).T
    return base + sp


def _gatv2(x, src, dst, edge_attr, Wl, Wr, We, att, bias):
    n = x.shape[0]
    xl = x @ Wl
    xr = x @ Wr
    e = jax.nn.leaky_relu(xl[src] + xr[dst] + edge_attr @ We, 0.2)
    logit = e @ att
    m = jax.ops.segment_max(logit, dst, num_segments=n)
    m = jnp.where(jnp.isfinite(m), m, 0.0)
    a = jnp.exp(logit - m[dst])
    denom = jax.ops.segment_sum(a, dst, num_segments=n)
    a = a / (denom[dst] + 1e-16)
    out = jax.ops.segment_sum(xl[src] * a[:, None], dst, num_segments=n)
    return out + bias


def _add_kernel(a_ref, b_ref, o_ref):
    o_ref[...] = a_ref[...] + b_ref[...]


@jax.jit
def _pallas_add(a, b):
    return pl.pallas_call(
        _add_kernel,
        out_shape=jax.ShapeDtypeStruct(a.shape, a.dtype),
        grid=(10,),
        in_specs=[pl.BlockSpec((a.shape[0] // 10, a.shape[1]), lambda i: (i, 0))] * 2,
        out_specs=pl.BlockSpec((a.shape[0] // 10, a.shape[1]), lambda i: (i, 0)),
    )(a, b)


def kernel(x, edge_attr, W0l, W0r, W0e, a0, b0, Wls, Wrs, Wes, atts, biases, kan_base, kan_spline, kan_base_last, kan_spline_last, edge_index):
    src, dst = edge_index[0], edge_index[1]
    h = _gatv2(x, src, dst, edge_attr, W0l, W0r, W0e, a0, b0)
    h = jax.nn.leaky_relu(h, 0.01)
    for i in range(4):
        res = h
        h = _gatv2(h, src, dst, edge_attr, Wls[i], Wrs[i], Wes[i], atts[i], biases[i])
        h = jax.nn.leaky_relu(h, 0.01)
        h = h + res
    h = _gatv2(h, src, dst, edge_attr, Wls[4], Wrs[4], Wes[4], atts[4], biases[4])
    grid = _make_grid(H)
    for i in range(3):
        h = _kan_linear(h, kan_base[i], kan_spline[i], grid)
    base = jax.nn.silu(h) @ kan_base_last.T
    sp = _b_splines(h, grid).reshape(h.shape[0], -1) @ kan_spline_last.reshape(KAN_OUT, -1).T
    return _pallas_add(base, sp)


# speedup vs baseline: 1.2326x; 1.2326x over previous
import functools

import jax
import jax.numpy as jnp
from jax import lax
from jax.experimental import pallas as pl

N = 10000
E = 320000
F_IN = 128
H = 256
D_E = 4
KAN_OUT = 100
GS = 5
SO = 3

NPAD = 10240
RB = 1024
KOUT_PAD = 128

_KNOT_H = 2.0 / GS
_KNOT0 = -SO * _KNOT_H - 1.0
_NKNOT = GS + 2 * SO + 1


def _knot(t):
    return _KNOT0 + _KNOT_H * t


def _splines8(x):
    bases = [
        jnp.where((x >= _knot(j)) & (x < _knot(j + 1)), 1.0, 0.0).astype(jnp.float32)
        for j in range(_NKNOT - 1)
    ]
    for k in range(1, SO + 1):
        inv = 1.0 / (k * _KNOT_H)
        bases = [
            (x - _knot(j)) * inv * bases[j]
            + (_knot(j + k + 1) - x) * inv * bases[j + 1]
            for j in range(len(bases) - 1)
        ]
    return bases


def _lrelu(v, s):
    return jnp.where(v >= 0, v, s * v)


def _finish_h(acc, den, bias, res, relu, resid):
    h = acc / (den + 1e-16) + bias
    if relu:
        h = _lrelu(h, 0.01)
    if resid:
        h = h + res
    return h


def _mm2_first_kernel(h_ref, wl_ref, wr_ref, xl_ref, xr_ref):
    h = h_ref[...]
    xl_ref[...] = jnp.dot(h, wl_ref[...], preferred_element_type=jnp.float32)
    xr_ref[...] = jnp.dot(h, wr_ref[...], preferred_element_type=jnp.float32)


def _mm2_first(h, wl, wr):
    return pl.pallas_call(
        _mm2_first_kernel,
        grid=(NPAD // RB,),
        in_specs=[
            pl.BlockSpec((RB, h.shape[1]), lambda i: (i, 0)),
            pl.BlockSpec((h.shape[1], H), lambda i: (0, 0)),
            pl.BlockSpec((h.shape[1], H), lambda i: (0, 0)),
        ],
        out_specs=[
            pl.BlockSpec((RB, H), lambda i: (i, 0)),
            pl.BlockSpec((RB, H), lambda i: (i, 0)),
        ],
        out_shape=[
            jax.ShapeDtypeStruct((NPAD, H), jnp.float32),
            jax.ShapeDtypeStruct((NPAD, H), jnp.float32),
        ],
    )(h, wl, wr)


def _act_mm2_kernel(acc_ref, den_ref, bias_ref, res_ref, wl_ref, wr_ref,
                    h_ref, xl_ref, xr_ref, *, relu, resid):
    h = _finish_h(acc_ref[...], den_ref[...], bias_ref[...],
                  res_ref[...] if resid else None, relu, resid)
    h_ref[...] = h
    xl_ref[...] = jnp.dot(h, wl_ref[...], preferred_element_type=jnp.float32)
    xr_ref[...] = jnp.dot(h, wr_ref[...], preferred_element_type=jnp.float32)


def _act_mm2(acc, den, bias, res, wl, wr, relu, resid):
    return pl.pallas_call(
        functools.partial(_act_mm2_kernel, relu=relu, resid=resid),
        grid=(NPAD // RB,),
        in_specs=[
            pl.BlockSpec((RB, H), lambda i: (i, 0)),
            pl.BlockSpec((RB, 1), lambda i: (i, 0)),
            pl.BlockSpec((1, H), lambda i: (0, 0)),
            pl.BlockSpec((RB, H), lambda i: (i, 0)),
            pl.BlockSpec((H, H), lambda i: (0, 0)),
            pl.BlockSpec((H, H), lambda i: (0, 0)),
        ],
        out_specs=[
            pl.BlockSpec((RB, H), lambda i: (i, 0)),
            pl.BlockSpec((RB, H), lambda i: (i, 0)),
            pl.BlockSpec((RB, H), lambda i: (i, 0)),
        ],
        out_shape=[
            jax.ShapeDtypeStruct((NPAD, H), jnp.float32),
            jax.ShapeDtypeStruct((NPAD, H), jnp.float32),
            jax.ShapeDtypeStruct((NPAD, H), jnp.float32),
        ],
    )(acc, den, bias, res, wl, wr)


def _kan_kernel(acc_ref, den_ref, bias_ref, bT_ref, sT_ref, blT_ref, slT_ref, out_ref):
    h = _finish_h(acc_ref[...], den_ref[...], bias_ref[...], None, False, False)
    for i in range(3):
        s = jnp.dot(h * jax.nn.sigmoid(h), bT_ref[i],
                    preferred_element_type=jnp.float32)
        for j, b in enumerate(_splines8(h)):
            s = s + jnp.dot(b, sT_ref[i, j], preferred_element_type=jnp.float32)
        h = s
    s = jnp.dot(h * jax.nn.sigmoid(h), blT_ref[...],
                preferred_element_type=jnp.float32)
    for j, b in enumerate(_splines8(h)):
        s = s + jnp.dot(b, slT_ref[j], preferred_element_type=jnp.float32)
    out_ref[...] = s


def _kan_stack(acc, den, bias, bT, sT, blT, slT):
    return pl.pallas_call(
        _kan_kernel,
        grid=(NPAD // RB,),
        in_specs=[
            pl.BlockSpec((RB, H), lambda i: (i, 0)),
            pl.BlockSpec((RB, 1), lambda i: (i, 0)),
            pl.BlockSpec((1, H), lambda i: (0, 0)),
            pl.BlockSpec((3, H, H), lambda i: (0, 0, 0)),
            pl.BlockSpec((3, 8, H, H), lambda i: (0, 0, 0, 0)),
            pl.BlockSpec((H, KOUT_PAD), lambda i: (0, 0)),
            pl.BlockSpec((8, H, KOUT_PAD), lambda i: (0, 0, 0)),
        ],
        out_specs=pl.BlockSpec((RB, KOUT_PAD), lambda i: (i, 0)),
        out_shape=jax.ShapeDtypeStruct((NPAD, KOUT_PAD), jnp.float32),
    )(acc, den, bias, bT, sT, blT, slT)


def _edge_xla(xl, xr, src, dst, edge_attr, we, att):
    logit = _lrelu(xl[src] + xr[dst] + edge_attr @ we, 0.2) @ att
    m = jax.ops.segment_max(logit, dst, num_segments=NPAD)
    m = jnp.where(jnp.isfinite(m), m, 0.0)
    a = jnp.exp(logit - m[dst])
    den = jax.ops.segment_sum(a, dst, num_segments=NPAD)
    acc = jax.ops.segment_sum(xl[src] * a[:, None], dst, num_segments=NPAD)
    return acc, den[:, None]


def kernel(x, edge_attr, W0l, W0r, W0e, a0, b0, Wls, Wrs, Wes, atts, biases,
           kan_base, kan_spline, kan_base_last, kan_spline_last, edge_index):
    src, dst = edge_index[0], edge_index[1]
    xp = jnp.pad(x, ((0, NPAD - N), (0, 0)))

    xl, xr = _mm2_first(xp, W0l, W0r)
    acc, den = _edge_xla(xl, xr, src, dst, edge_attr, W0e, a0)
    bias = b0[None, :]
    h = None
    for i in range(5):
        resid = 0 < i
        res = h if resid else jnp.zeros((NPAD, H), jnp.float32)
        h, xl, xr = _act_mm2(acc, den, bias, res, Wls[i], Wrs[i], True, resid)
        acc, den = _edge_xla(xl, xr, src, dst, edge_attr, Wes[i], atts[i])
        bias = biases[i][None, :]

    bT = jnp.transpose(kan_base, (0, 2, 1))
    sT = jnp.transpose(kan_spline, (0, 3, 2, 1))
    blT = jnp.pad(kan_base_last.T, ((0, 0), (0, KOUT_PAD - KAN_OUT)))
    slT = jnp.pad(jnp.transpose(kan_spline_last, (2, 1, 0)),
                  ((0, 0), (0, 0), (0, KOUT_PAD - KAN_OUT)))
    out = _kan_stack(acc, den, bias, bT, sT, blT, slT)
    return out[:N, :KAN_OUT]


# speedup vs baseline: 1.6365x; 1.3276x over previous
import functools

import jax
import jax.numpy as jnp
from jax import lax
from jax.experimental import pallas as pl
from jax.experimental.pallas import tpu as pltpu
from jax.experimental.pallas import tpu_sc as plsc

N = 10000
E = 320000
F_IN = 128
H = 256
D_E = 4
KAN_OUT = 100
GS = 5
SO = 3

NPAD = 10240
RB = 1024
KOUT_PAD = 128

_KNOT_H = 2.0 / GS
_KNOT0 = -SO * _KNOT_H - 1.0
_NKNOT = GS + 2 * SO + 1


def _knot(t):
    return _KNOT0 + _KNOT_H * t


def _splines8(x):
    bases = [
        jnp.where((x >= _knot(j)) & (x < _knot(j + 1)), 1.0, 0.0).astype(jnp.float32)
        for j in range(_NKNOT - 1)
    ]
    for k in range(1, SO + 1):
        inv = 1.0 / (k * _KNOT_H)
        bases = [
            (x - _knot(j)) * inv * bases[j]
            + (_knot(j + k + 1) - x) * inv * bases[j + 1]
            for j in range(len(bases) - 1)
        ]
    return bases


def _lrelu(v, s):
    return jnp.where(v >= 0, v, s * v)


def _finish_h(acc, den, bias, res, relu, resid):
    h = acc / (den + 1e-16) + bias
    if relu:
        h = _lrelu(h, 0.01)
    if resid:
        h = h + res
    return h


def _mm2_first_kernel(h_ref, wl_ref, wr_ref, xl_ref, xr_ref):
    h = h_ref[...]
    xl_ref[...] = jnp.dot(h, wl_ref[...], preferred_element_type=jnp.float32)
    xr_ref[...] = jnp.dot(h, wr_ref[...], preferred_element_type=jnp.float32)


def _mm2_first(h, wl, wr):
    return pl.pallas_call(
        _mm2_first_kernel,
        grid=(NPAD // RB,),
        in_specs=[
            pl.BlockSpec((RB, h.shape[1]), lambda i: (i, 0)),
            pl.BlockSpec((h.shape[1], H), lambda i: (0, 0)),
            pl.BlockSpec((h.shape[1], H), lambda i: (0, 0)),
        ],
        out_specs=[
            pl.BlockSpec((RB, H), lambda i: (i, 0)),
            pl.BlockSpec((RB, H), lambda i: (i, 0)),
        ],
        out_shape=[
            jax.ShapeDtypeStruct((NPAD, H), jnp.float32),
            jax.ShapeDtypeStruct((NPAD, H), jnp.float32),
        ],
    )(h, wl, wr)


def _act_mm2_kernel(acc_ref, den_ref, bias_ref, res_ref, wl_ref, wr_ref,
                    h_ref, xl_ref, xr_ref, *, relu, resid):
    h = _finish_h(acc_ref[...], den_ref[...], bias_ref[...],
                  res_ref[...] if resid else None, relu, resid)
    h_ref[...] = h
    xl_ref[...] = jnp.dot(h, wl_ref[...], preferred_element_type=jnp.float32)
    xr_ref[...] = jnp.dot(h, wr_ref[...], preferred_element_type=jnp.float32)


def _act_mm2(acc, den, bias, res, wl, wr, relu, resid):
    return pl.pallas_call(
        functools.partial(_act_mm2_kernel, relu=relu, resid=resid),
        grid=(NPAD // RB,),
        in_specs=[
            pl.BlockSpec((RB, H), lambda i: (i, 0)),
            pl.BlockSpec((RB, 1), lambda i: (i, 0)),
            pl.BlockSpec((1, H), lambda i: (0, 0)),
            pl.BlockSpec((RB, H), lambda i: (i, 0)),
            pl.BlockSpec((H, H), lambda i: (0, 0)),
            pl.BlockSpec((H, H), lambda i: (0, 0)),
        ],
        out_specs=[
            pl.BlockSpec((RB, H), lambda i: (i, 0)),
            pl.BlockSpec((RB, H), lambda i: (i, 0)),
            pl.BlockSpec((RB, H), lambda i: (i, 0)),
        ],
        out_shape=[
            jax.ShapeDtypeStruct((NPAD, H), jnp.float32),
            jax.ShapeDtypeStruct((NPAD, H), jnp.float32),
            jax.ShapeDtypeStruct((NPAD, H), jnp.float32),
        ],
    )(acc, den, bias, res, wl, wr)


def _kan_kernel(acc_ref, den_ref, bias_ref, bT_ref, sT_ref, blT_ref, slT_ref, out_ref):
    h = _finish_h(acc_ref[...], den_ref[...], bias_ref[...], None, False, False)
    for i in range(3):
        s = jnp.dot(h * jax.nn.sigmoid(h), bT_ref[i],
                    preferred_element_type=jnp.float32)
        for j, b in enumerate(_splines8(h)):
            s = s + jnp.dot(b, sT_ref[i, j], preferred_element_type=jnp.float32)
        h = s
    s = jnp.dot(h * jax.nn.sigmoid(h), blT_ref[...],
                preferred_element_type=jnp.float32)
    for j, b in enumerate(_splines8(h)):
        s = s + jnp.dot(b, slT_ref[j], preferred_element_type=jnp.float32)
    out_ref[...] = s


def _kan_stack(acc, den, bias, bT, sT, blT, slT):
    return pl.pallas_call(
        _kan_kernel,
        grid=(NPAD // RB,),
        in_specs=[
            pl.BlockSpec((RB, H), lambda i: (i, 0)),
            pl.BlockSpec((RB, 1), lambda i: (i, 0)),
            pl.BlockSpec((1, H), lambda i: (0, 0)),
            pl.BlockSpec((3, H, H), lambda i: (0, 0, 0)),
            pl.BlockSpec((3, 8, H, H), lambda i: (0, 0, 0, 0)),
            pl.BlockSpec((H, KOUT_PAD), lambda i: (0, 0)),
            pl.BlockSpec((8, H, KOUT_PAD), lambda i: (0, 0, 0)),
        ],
        out_specs=pl.BlockSpec((RB, KOUT_PAD), lambda i: (i, 0)),
        out_shape=jax.ShapeDtypeStruct((NPAD, KOUT_PAD), jnp.float32),
    )(acc, den, bias, bT, sT, blT, slT)


NW = 32
NB = NPAD // NW
NBT = NB + 1
CAP = 12288
GRP = 128
CH = 2000

def _mesh():
    return plsc.VectorSubcoreMesh(core_axis_name="c", subcore_axis_name="s")


def _wid():
    return lax.axis_index("s") * 2 + lax.axis_index("c")


def _partition_body(src_hbm, dst_hbm, e0_hbm, e1_hbm, e2_hbm, e3_hbm,
                    srcb_hbm, dstb_hbm, b0_hbm, b1_hbm, b2_hbm, b3_hbm, cnt_hbm,
                    srcl, dstl, l0, l1, l2, l3,
                    srct, dstt, t0, t1, t2, t3, cntv):
    w = _wid()
    lo = w * NB

    def blk_body(blk, cnt):
        off = blk * CH
        pltpu.sync_copy(src_hbm.at[pl.ds(off, CH)], srct)
        pltpu.sync_copy(dst_hbm.at[pl.ds(off, CH)], dstt)
        pltpu.sync_copy(e0_hbm.at[pl.ds(off, CH)], t0)
        pltpu.sync_copy(e1_hbm.at[pl.ds(off, CH)], t1)
        pltpu.sync_copy(e2_hbm.at[pl.ds(off, CH)], t2)
        pltpu.sync_copy(e3_hbm.at[pl.ds(off, CH)], t3)

        def v_body(v, cnt):
            sl = pl.ds(v * 16, 16)
            d = dstt[sl]
            msk = (d >= lo) & (d < lo + NB)
            mi = msk.astype(jnp.int32)
            pos = cnt + plsc.cumsum(mi) - 1
            plsc.store_scatter(srcl, [pos], srct[sl], mask=msk)
            plsc.store_scatter(dstl, [pos], d, mask=msk)
            plsc.store_scatter(l0, [pos], t0[sl], mask=msk)
            plsc.store_scatter(l1, [pos], t1[sl], mask=msk)
            plsc.store_scatter(l2, [pos], t2[sl], mask=msk)
            plsc.store_scatter(l3, [pos], t3[sl], mask=msk)
            return cnt + jnp.sum(mi, axis=0)

        return lax.fori_loop(0, CH // 16, v_body, cnt)

    cnt = lax.fori_loop(0, E // CH, blk_body, jnp.int32(0))

    zi = jnp.zeros((16,), jnp.int32)
    zf = jnp.zeros((16,), jnp.float32)
    sent_d = jnp.full((16,), lo + NB, jnp.int32)
    for t in range(GRP // 16):
        sl = pl.ds(cnt + 16 * t, 16)
        srcl[sl] = zi
        dstl[sl] = sent_d
        l0[sl] = zf
        l1[sl] = zf
        l2[sl] = zf
        l3[sl] = zf
    cnt_pad = ((cnt + GRP - 1) // GRP) * GRP

    pltpu.sync_copy(srcl, srcb_hbm.at[w])
    pltpu.sync_copy(dstl, dstb_hbm.at[w])
    pltpu.sync_copy(l0, b0_hbm.at[w])
    pltpu.sync_copy(l1, b1_hbm.at[w])
    pltpu.sync_copy(l2, b2_hbm.at[w])
    pltpu.sync_copy(l3, b3_hbm.at[w])
    cntv[...] = jnp.full((16,), cnt_pad, jnp.int32)
    pltpu.sync_copy(cntv, cnt_hbm.at[w])


@jax.jit
def _partition(src, dst, e0, e1, e2, e3):
    f = pl.kernel(
        _partition_body,
        out_type=[
            jax.ShapeDtypeStruct((NW, CAP), jnp.int32),
            jax.ShapeDtypeStruct((NW, CAP), jnp.int32),
            jax.ShapeDtypeStruct((NW, CAP), jnp.float32),
            jax.ShapeDtypeStruct((NW, CAP), jnp.float32),
            jax.ShapeDtypeStruct((NW, CAP), jnp.float32),
            jax.ShapeDtypeStruct((NW, CAP), jnp.float32),
            jax.ShapeDtypeStruct((NW, 16), jnp.int32),
        ],
        mesh=_mesh(),
        compiler_params=pltpu.CompilerParams(needs_layout_passes=False),
        scratch_types=[
            pltpu.VMEM((CAP,), jnp.int32),
            pltpu.VMEM((CAP,), jnp.int32),
            pltpu.VMEM((CAP,), jnp.float32),
            pltpu.VMEM((CAP,), jnp.float32),
            pltpu.VMEM((CAP,), jnp.float32),
            pltpu.VMEM((CAP,), jnp.float32),
            pltpu.VMEM((CH,), jnp.int32),
            pltpu.VMEM((CH,), jnp.int32),
            pltpu.VMEM((CH,), jnp.float32),
            pltpu.VMEM((CH,), jnp.float32),
            pltpu.VMEM((CH,), jnp.float32),
            pltpu.VMEM((CH,), jnp.float32),
            pltpu.VMEM((16,), jnp.int32),
        ],
    )
    return f(src, dst, e0, e1, e2, e3)


def _edge_body(xl_hbm, xr_hbm, srcb, dstb, b0, b1, b2, b3, cntb, we_hbm, att_hbm,
               acc_hbm, den_hbm, lg_hbm,
               tab, xlrows, m_tab, den_tab,
               srcv, dstv, v0, v1, v2, v3, wv, lgv,
               sem, wesp, attsp, cntsp, dstsp, we_s, att_s, dst_s, cnt_s):
    w = _wid()
    sid = lax.axis_index("s")
    lo = w * NB
    lanes = lax.iota(jnp.int32, 16)

    pltpu.sync_copy(we_hbm, wesp.at[sid])
    pltpu.sync_copy(wesp.at[sid], we_s)
    pltpu.sync_copy(att_hbm, attsp.at[sid])
    pltpu.sync_copy(attsp.at[sid], att_s)
    pltpu.sync_copy(cntb.at[w], cntsp.at[sid])
    pltpu.sync_copy(cntsp.at[sid], cnt_s)
    ngrp = cnt_s[0] // GRP

    pltpu.sync_copy(xr_hbm.at[pl.ds(lo, NB)], tab.at[pl.ds(0, NB)])
    zf = jnp.zeros((16,), jnp.float32)
    for c in range(H // 16):
        tab[NB, pl.ds(c * 16, 16)] = zf
    neg = jnp.full((16,), -1e30, jnp.float32)
    for r in range(NBT // 16 + 1):
        m_tab[pl.ds(r * 16, 16)] = neg

    def fetch_group(g):
        base = pl.ds(g * GRP, GRP)
        pltpu.sync_copy(srcb.at[w, base], srcv)
        pltpu.sync_copy(dstb.at[w, base], dstv)
        pltpu.async_copy(xl_hbm.at[srcv], xlrows, sem).wait()

    def pass_a(g, carry):
        fetch_group(g)
        base = pl.ds(g * GRP, GRP)
        pltpu.sync_copy(b0.at[w, base], v0)
        pltpu.sync_copy(b1.at[w, base], v1)
        pltpu.sync_copy(b2.at[w, base], v2)
        pltpu.sync_copy(b3.at[w, base], v3)
        for j in range(GRP // 16):
            sl = pl.ds(j * 16, 16)
            dl = dstv[sl] - lo
            ea0, ea1, ea2, ea3 = v0[sl], v1[sl], v2[sl], v3[sl]
            ei = lanes + (j * 16)

            def col_blk(cb, lg):
                for u in range(16):
                    c = cb * 16 + u
                    cc = jnp.full((16,), c, jnp.int32)
                    xlc = plsc.load_gather(xlrows, [ei, cc])
                    xrc = plsc.load_gather(tab, [dl, cc])
                    v = (xlc + xrc + ea0 * we_s[0, c] + ea1 * we_s[1, c]
                         + ea2 * we_s[2, c] + ea3 * we_s[3, c])
                    lg = lg + jnp.maximum(v, 0.2 * v) * att_s[c]
                return lg

            lg = lax.fori_loop(0, H // 16, col_blk, jnp.zeros((16,), jnp.float32))
            lgv[sl] = lg
            mold = plsc.load_gather(m_tab, [dl])
            plsc.store_scatter(m_tab, [dl], jnp.maximum(mold, lg))
        pltpu.sync_copy(lgv, lg_hbm.at[w, base])
        return carry

    lax.fori_loop(0, ngrp, pass_a, jnp.int32(0))

    def zero_row(r, carry):
        for c in range(H // 16):
            tab[r, pl.ds(c * 16, 16)] = zf
        den_tab[pl.ds(r * 16, 16)] = zf
        return carry

    lax.fori_loop(0, NBT, zero_row, jnp.int32(0))

    def pass_b(g, carry):
        fetch_group(g)
        base = pl.ds(g * GRP, GRP)
        pltpu.sync_copy(lg_hbm.at[w, base], lgv)
        pltpu.sync_copy(dstb.at[w, base], dstsp.at[sid])
        pltpu.sync_copy(dstsp.at[sid], dst_s)
        for j in range(GRP // 16):
            sl = pl.ds(j * 16, 16)
            dl = dstv[sl] - lo
            mv = plsc.load_gather(m_tab, [dl])
            wv[sl] = jnp.exp(jnp.clip(lgv[sl] - mv, -100.0, 80.0))
        def edge_body(e, carry2):
            drow = dst_s[e] - lo
            wspl = plsc.load_gather(wv, [jnp.full((16,), e, jnp.int32)])
            for c in range(H // 16):
                cs = pl.ds(c * 16, 16)
                tab[drow, cs] = tab[drow, cs] + wspl * xlrows[e, cs]
            ds16 = pl.ds(drow * 16, 16)
            den_tab[ds16] = den_tab[ds16] + wspl
            return carry2

        return lax.fori_loop(0, GRP, edge_body, carry)

    lax.fori_loop(0, ngrp, pass_b, jnp.int32(0))

    pltpu.sync_copy(tab.at[pl.ds(0, NB)], acc_hbm.at[pl.ds(lo, NB)])
    pltpu.sync_copy(den_tab.at[pl.ds(0, NB * 16)], den_hbm.at[pl.ds(lo * 16, NB * 16)])


@jax.jit
def _edge_sc(xl, xr, srcb, dstb, b0, b1, b2, b3, cntb, we, att):
    f = pl.kernel(
        _edge_body,
        out_type=[
            jax.ShapeDtypeStruct((NPAD, H), jnp.float32),
            jax.ShapeDtypeStruct((NPAD * 16,), jnp.float32),
            jax.ShapeDtypeStruct((NW, CAP), jnp.float32),
        ],
        mesh=_mesh(),
        compiler_params=pltpu.CompilerParams(needs_layout_passes=False),
        scratch_types=[
            pltpu.VMEM((NBT, H), jnp.float32),
            pltpu.VMEM((GRP, H), jnp.float32),
            pltpu.VMEM((NBT + 16,), jnp.float32),
            pltpu.VMEM((NBT * 16,), jnp.float32),
            pltpu.VMEM((GRP,), jnp.int32),
            pltpu.VMEM((GRP,), jnp.int32),
            pltpu.VMEM((GRP,), jnp.float32),
            pltpu.VMEM((GRP,), jnp.float32),
            pltpu.VMEM((GRP,), jnp.float32),
            pltpu.VMEM((GRP,), jnp.float32),
            pltpu.VMEM((GRP,), jnp.float32),
            pltpu.VMEM((GRP,), jnp.float32),
            pltpu.SemaphoreType.DMA,
            pltpu.VMEM_SHARED((16, D_E, H), jnp.float32),
            pltpu.VMEM_SHARED((16, H), jnp.float32),
            pltpu.VMEM_SHARED((16, 16), jnp.int32),
            pltpu.VMEM_SHARED((16, GRP), jnp.int32),
            pltpu.SMEM((D_E, H), jnp.float32),
            pltpu.SMEM((H,), jnp.float32),
            pltpu.SMEM((GRP,), jnp.int32),
            pltpu.SMEM((16,), jnp.int32),
        ],
    )
    acc, den, _ = f(xl, xr, srcb, dstb, b0, b1, b2, b3, cntb, we, att)
    return acc, den.reshape(NPAD, 16)[:, :1]


def _edge_xla(xl, xr, src, dst, edge_attr, we, att):
    logit = _lrelu(xl[src] + xr[dst] + edge_attr @ we, 0.2) @ att
    m = jax.ops.segment_max(logit, dst, num_segments=NPAD)
    m = jnp.where(jnp.isfinite(m), m, 0.0)
    a = jnp.exp(logit - m[dst])
    den = jax.ops.segment_sum(a, dst, num_segments=NPAD)
    acc = jax.ops.segment_sum(xl[src] * a[:, None], dst, num_segments=NPAD)
    return acc, den[:, None]


def kernel(x, edge_attr, W0l, W0r, W0e, a0, b0, Wls, Wrs, Wes, atts, biases,
           kan_base, kan_spline, kan_base_last, kan_spline_last, edge_index):
    src, dst = edge_index[0], edge_index[1]
    xp = jnp.pad(x, ((0, NPAD - N), (0, 0)))
    ecols = [jnp.asarray(edge_attr[:, j]) for j in range(D_E)]
    srcb, dstb, c0, c1, c2, c3, cntb = _partition(src, dst, *ecols)
    bkt = (srcb, dstb, c0, c1, c2, c3, cntb)

    xl, xr = _mm2_first(xp, W0l, W0r)
    acc, den = _edge_sc(xl, xr, *bkt, W0e, a0)
    bias = b0[None, :]
    h = None
    for i in range(5):
        resid = 0 < i
        res = h if resid else jnp.zeros((NPAD, H), jnp.float32)
        h, xl, xr = _act_mm2(acc, den, bias, res, Wls[i], Wrs[i], True, resid)
        acc, den = _edge_sc(xl, xr, *bkt, Wes[i], atts[i])
        bias = biases[i][None, :]

    bT = jnp.transpose(kan_base, (0, 2, 1))
    sT = jnp.transpose(kan_spline, (0, 3, 2, 1))
    blT = jnp.pad(kan_base_last.T, ((0, 0), (0, KOUT_PAD - KAN_OUT)))
    slT = jnp.pad(jnp.transpose(kan_spline_last, (2, 1, 0)),
                  ((0, 0), (0, 0), (0, KOUT_PAD - KAN_OUT)))
    out = _kan_stack(acc, den, bias, bT, sT, blT, slT)
    return out[:N, :KAN_OUT]
